# baseline (speedup 1.0000x reference)
import functools

import jax
import jax.numpy as jnp
from jax import lax
from jax.experimental import pallas as pl
from jax.experimental.pallas import tpu as pltpu
from jax.experimental.pallas import tpu_sc as plsc

_MAX_VAL = 2097151
_NC = 2
_NS = 16
_NW = _NC * _NS
_L = 16
_NSUB = 8


@functools.lru_cache(maxsize=None)
def _build(n, interpret=False):
    assert n % (_NW * 4 * _L * _NSUB) == 0, n
    w = n // 4
    w_tile = w // _NW
    e_tile = n // _NW
    w_sub = w_tile // _NSUB
    e_sub = e_tile // _NSUB
    n_flush = 128
    assert w_sub % (_L * n_flush) == 0 or (w_sub % _L == 0 and w_sub // _L <= n_flush)

    mesh = plsc.VectorSubcoreMesh(
        core_axis_name="c", subcore_axis_name="s",
        num_cores=_NC, num_subcores=_NS,
    )
    cparams = pltpu.CompilerParams(needs_layout_passes=False)

    @functools.partial(
        pl.kernel,
        out_type=jax.ShapeDtypeStruct((_NW, _L), jnp.int32),
        mesh=mesh,
        scratch_types=[
            pltpu.VMEM((w_sub,), jnp.int32),
            pltpu.VMEM((_L,), jnp.int32),
        ],
        compiler_params=cparams,
        interpret=interpret,
    )
    def _sums_kernel(words_hbm, out_hbm, buf, outv):
        wid = lax.axis_index("c") * _NS + lax.axis_index("s")
        base = wid * w_tile
        n_vregs = w_sub // _L
        n_blk = max(n_vregs // n_flush, 1)
        per_blk = n_vregs // n_blk

        def sub_body(sub, acc32):
            pltpu.sync_copy(words_hbm.at[pl.ds(base + sub * w_sub, w_sub)], buf)

            def blk(b, acc32):
                def it(i, accb):
                    return accb + buf[pl.ds((b * per_blk + i) * _L, _L)]

                accb = lax.fori_loop(0, per_blk, it, jnp.zeros((_L,), jnp.int32))
                a0 = accb & 0xFF
                a1 = (accb >> 8) & 0xFF
                a2 = (accb >> 16) & 0xFF
                a3 = (accb >> 24) & 0xFF
                return acc32 + a0 + a1 + a2 + a3

            return lax.fori_loop(0, n_blk, blk, acc32)

        acc32 = lax.fori_loop(0, _NSUB, sub_body, jnp.zeros((_L,), jnp.int32))
        outv[...] = acc32
        pltpu.sync_copy(outv, out_hbm.at[wid])

    @functools.partial(
        pl.kernel,
        out_type=jax.ShapeDtypeStruct((n,), jnp.int32),
        mesh=mesh,
        scratch_types=[
            pltpu.VMEM((w_sub,), jnp.int32),
            pltpu.VMEM((e_sub,), jnp.int32),
            pltpu.VMEM((_NW, _L), jnp.int32),
        ],
        compiler_params=cparams,
        interpret=interpret,
    )
    def _scan_kernel(words_hbm, sums_hbm, out_hbm, wbuf, obuf, sums_v):
        wid = lax.axis_index("c") * _NS + lax.axis_index("s")
        base_w = wid * w_tile
        base_e = wid * e_tile
        pltpu.sync_copy(sums_hbm, sums_v)

        def acc_row(wp, carryv):
            m = (wp < wid).astype(jnp.int32)
            return carryv + sums_v[wp] * m

        carryv = lax.fori_loop(0, _NW, acc_row, jnp.zeros((_L,), jnp.int32))
        carry0 = jnp.sum(carryv) - 1

        lanes4 = lax.iota(jnp.int32, _L) * 4

        def sub_body(sub, carry):
            pltpu.sync_copy(words_hbm.at[pl.ds(base_w + sub * w_sub, w_sub)], wbuf)

            def it(i, carry):
                word = wbuf[pl.ds(i * _L, _L)]
                p = word * 0x01010101
                t = (p >> 24) & 0xFF
                incl = plsc.cumsum(t)
                basev = (incl - t) + carry
                opos = lanes4 + i * (4 * _L)
                for j in range(4):
                    bj = (p >> (8 * j)) & 0xFF
                    o = jnp.minimum(jnp.maximum(basev + bj, 0), _MAX_VAL)
                    plsc.store_scatter(obuf, [opos + j], o)
                return carry + jnp.sum(t)

            carry = lax.fori_loop(0, w_sub // _L, it, carry)
            pltpu.sync_copy(obuf, out_hbm.at[pl.ds(base_e + sub * e_sub, e_sub)])
            return carry

        lax.fori_loop(0, _NSUB, sub_body, carry0)

    def run(words):
        sums = _sums_kernel(words)
        return _scan_kernel(words, sums)

    return run


def kernel(mask_1d):
    n = mask_1d.shape[0]
    words = lax.bitcast_convert_type(
        mask_1d.astype(jnp.int8).reshape(n // 4, 4), jnp.int32
    )
    return _build(n)(words)


# speedup vs baseline: 9.6126x; 9.6126x over previous
import functools

import jax
import jax.numpy as jnp
from jax import lax
from jax.experimental import pallas as pl
from jax.experimental.pallas import tpu as pltpu
from jax.experimental.pallas import tpu_sc as plsc

_MAX_VAL = 2097151
_NC = 2
_NS = 16
_NW = _NC * _NS
_L = 16
_NSUB = 8


@functools.lru_cache(maxsize=None)
def _build(n, interpret=False):
    assert n % (_NW * 4 * _L * _NSUB) == 0, n
    w = n // 4
    w_tile = w // _NW
    e_tile = n // _NW
    w_sub = w_tile // _NSUB
    e_sub = e_tile // _NSUB
    n_flush = 128
    assert w_sub % (_L * n_flush) == 0 or (w_sub % _L == 0 and w_sub // _L <= n_flush)

    mesh = plsc.VectorSubcoreMesh(
        core_axis_name="c", subcore_axis_name="s",
        num_cores=_NC, num_subcores=_NS,
    )
    cparams = pltpu.CompilerParams(needs_layout_passes=False)

    @functools.partial(
        pl.kernel,
        out_type=jax.ShapeDtypeStruct((_NW, _L), jnp.int32),
        mesh=mesh,
        scratch_types=[
            pltpu.VMEM((w_sub * 4,), jnp.int8),
            pltpu.VMEM((_L,), jnp.int32),
        ],
        compiler_params=cparams,
        interpret=interpret,
    )
    def _sums_kernel(bytes_hbm, out_hbm, buf, outv):
        wid = lax.axis_index("c") * _NS + lax.axis_index("s")
        base = wid * w_tile
        n_vregs = w_sub // _L
        n_blk = max(n_vregs // n_flush, 1)
        per_blk = n_vregs // n_blk

        def sub_body(sub, acc32):
            pltpu.sync_copy(
                bytes_hbm.at[pl.ds((base + sub * w_sub) * 4, w_sub * 4)], buf
            )

            def blk(b, acc32):
                def it(i, accb):
                    v8 = buf[pl.ds((b * per_blk + i) * _L * 4, _L * 4)]
                    return accb + plsc.bitcast(v8, jnp.int32)

                accb = lax.fori_loop(0, per_blk, it, jnp.zeros((_L,), jnp.int32))
                a0 = accb & 0xFF
                a1 = (accb >> 8) & 0xFF
                a2 = (accb >> 16) & 0xFF
                a3 = (accb >> 24) & 0xFF
                return acc32 + a0 + a1 + a2 + a3

            return lax.fori_loop(0, n_blk, blk, acc32)

        acc32 = lax.fori_loop(0, _NSUB, sub_body, jnp.zeros((_L,), jnp.int32))
        outv[...] = acc32
        pltpu.sync_copy(outv, out_hbm.at[wid])

    @functools.partial(
        pl.kernel,
        out_type=jax.ShapeDtypeStruct((n,), jnp.int32),
        mesh=mesh,
        scratch_types=[
            pltpu.VMEM((w_sub * 4,), jnp.int8),
            pltpu.VMEM((e_sub,), jnp.int32),
            pltpu.VMEM((_NW, _L), jnp.int32),
        ],
        compiler_params=cparams,
        interpret=interpret,
    )
    def _scan_kernel(bytes_hbm, sums_hbm, out_hbm, wbuf, obuf, sums_v):
        wid = lax.axis_index("c") * _NS + lax.axis_index("s")
        base_w = wid * w_tile
        base_e = wid * e_tile
        pltpu.sync_copy(sums_hbm, sums_v)

        def acc_row(wp, carryv):
            m = (wp < wid).astype(jnp.int32)
            return carryv + sums_v[wp] * m

        carryv = lax.fori_loop(0, _NW, acc_row, jnp.zeros((_L,), jnp.int32))
        carry0 = jnp.sum(carryv) - 1

        lanes4 = lax.iota(jnp.int32, _L) * 4

        def sub_body(sub, carry):
            pltpu.sync_copy(
                bytes_hbm.at[pl.ds((base_w + sub * w_sub) * 4, w_sub * 4)], wbuf
            )

            def it(i, carry):
                word = plsc.bitcast(wbuf[pl.ds(i * _L * 4, _L * 4)], jnp.int32)
                p = word * 0x01010101
                t = (p >> 24) & 0xFF
                incl = plsc.cumsum(t)
                basev = (incl - t) + carry
                opos = lanes4 + i * (4 * _L)
                for j in range(4):
                    bj = (p >> (8 * j)) & 0xFF
                    o = jnp.minimum(jnp.maximum(basev + bj, 0), _MAX_VAL)
                    plsc.store_scatter(obuf, [opos + j], o)
                return carry + jnp.sum(t)

            carry = lax.fori_loop(0, w_sub // _L, it, carry)
            pltpu.sync_copy(obuf, out_hbm.at[pl.ds(base_e + sub * e_sub, e_sub)])
            return carry

        lax.fori_loop(0, _NSUB, sub_body, carry0)

    def run(mask_bytes):
        sums = _sums_kernel(mask_bytes)
        return _scan_kernel(mask_bytes, sums)

    return run


def kernel(mask_1d):
    n = mask_1d.shape[0]
    return _build(n)(mask_1d.astype(jnp.int8))


# speedup vs baseline: 10.2799x; 1.0694x over previous
import functools

import jax
import jax.numpy as jnp
from jax import lax
from jax.experimental import pallas as pl
from jax.experimental.pallas import tpu as pltpu
from jax.experimental.pallas import tpu_sc as plsc

_MAX_VAL = 2097151
_NC = 2
_NS = 16
_NW = _NC * _NS
_L = 16
_NSUB = 8


@functools.lru_cache(maxsize=None)
def _build(n):
    assert n % (_NW * 4 * _L * _NSUB) == 0, n
    e_tile = n // _NW
    e_sub = e_tile // _NSUB

    mesh = plsc.VectorSubcoreMesh(
        core_axis_name="c", subcore_axis_name="s",
        num_cores=_NC, num_subcores=_NS,
    )
    cparams = pltpu.CompilerParams(needs_layout_passes=False)

    @functools.partial(
        pl.kernel,
        out_type=jax.ShapeDtypeStruct((_NW, _L), jnp.int32),
        mesh=mesh,
        scratch_types=[
            pltpu.VMEM((e_sub,), jnp.int32),
            pltpu.VMEM((_L,), jnp.int32),
        ],
        compiler_params=cparams,
    )
    def _sums_kernel(mask_hbm, out_hbm, buf, outv):
        wid = lax.axis_index("c") * _NS + lax.axis_index("s")
        base = wid * e_tile

        def sub_body(sub, acc):
            pltpu.sync_copy(mask_hbm.at[pl.ds(base + sub * e_sub, e_sub)], buf)

            def it(i, acc):
                a = buf[pl.ds(i * 4 * _L, _L)]
                b = buf[pl.ds((i * 4 + 1) * _L, _L)]
                c = buf[pl.ds((i * 4 + 2) * _L, _L)]
                d = buf[pl.ds((i * 4 + 3) * _L, _L)]
                return acc + ((a + b) + (c + d))

            return lax.fori_loop(0, e_sub // (4 * _L), it, acc)

        acc = lax.fori_loop(0, _NSUB, sub_body, jnp.zeros((_L,), jnp.int32))
        outv[...] = acc
        pltpu.sync_copy(outv, out_hbm.at[wid])

    @functools.partial(
        pl.kernel,
        out_type=jax.ShapeDtypeStruct((n,), jnp.int32),
        mesh=mesh,
        scratch_types=[
            pltpu.VMEM((e_sub,), jnp.int32),
            pltpu.VMEM((e_sub,), jnp.int32),
            pltpu.VMEM((_NW, _L), jnp.int32),
        ],
        compiler_params=cparams,
    )
    def _scan_kernel(mask_hbm, sums_hbm, out_hbm, mbuf, obuf, sums_v):
        wid = lax.axis_index("c") * _NS + lax.axis_index("s")
        base = wid * e_tile
        pltpu.sync_copy(sums_hbm, sums_v)

        def acc_row(wp, carryv):
            m = (wp < wid).astype(jnp.int32)
            return carryv + sums_v[wp] * m

        carry0 = lax.fori_loop(0, _NW, acc_row, jnp.zeros((_L,), jnp.int32))
        carry0 = jnp.full((_L,), jnp.sum(carry0) - 1, jnp.int32)

        def sub_body(sub, carry):
            pltpu.sync_copy(mask_hbm.at[pl.ds(base + sub * e_sub, e_sub)], mbuf)

            def it(i, carry):
                v0 = mbuf[pl.ds(i * 4 * _L, _L)]
                v1 = mbuf[pl.ds((i * 4 + 1) * _L, _L)]
                v2 = mbuf[pl.ds((i * 4 + 2) * _L, _L)]
                v3 = mbuf[pl.ds((i * 4 + 3) * _L, _L)]
                packed = v0 + (v1 << 8) + ((v2 << 16) + (v3 << 24))
                incl = plsc.cumsum(packed)
                s = jnp.sum(packed)
                cbefore = s * 0x01010100
                b0 = incl & 0xFF
                b1 = (incl >> 8) & 0xFF
                b2 = (incl >> 16) & 0xFF
                b3 = incl >> 24
                o0 = carry + b0
                o1 = (carry + ((cbefore >> 8) & 0xFF)) + b1
                o2 = (carry + ((cbefore >> 16) & 0xFF)) + b2
                o3 = (carry + (cbefore >> 24)) + b3
                zero = jnp.int32(0)
                obuf[pl.ds(i * 4 * _L, _L)] = jnp.minimum(
                    jnp.maximum(o0, zero), _MAX_VAL)
                obuf[pl.ds((i * 4 + 1) * _L, _L)] = jnp.minimum(
                    jnp.maximum(o1, zero), _MAX_VAL)
                obuf[pl.ds((i * 4 + 2) * _L, _L)] = jnp.minimum(
                    jnp.maximum(o2, zero), _MAX_VAL)
                obuf[pl.ds((i * 4 + 3) * _L, _L)] = jnp.minimum(
                    jnp.maximum(o3, zero), _MAX_VAL)
                total = (cbefore >> 24) + (s >> 24)
                return carry + total

            carry = lax.fori_loop(0, e_sub // (4 * _L), it, carry)
            pltpu.sync_copy(obuf, out_hbm.at[pl.ds(base + sub * e_sub, e_sub)])
            return carry

        lax.fori_loop(0, _NSUB, sub_body, carry0)

    def run(mask_i32):
        sums = _sums_kernel(mask_i32)
        return _scan_kernel(mask_i32, sums)

    return run


def kernel(mask_1d):
    n = mask_1d.shape[0]
    return _build(n)(mask_1d.astype(jnp.int32))
